# baseline (speedup 1.0000x reference)
import jax
import jax.numpy as jnp
from jax import lax
from jax.experimental import pallas as pl
from jax.experimental.pallas import tpu as pltpu

_CHUNK = 200


def _policy_kernel(state_ref, W_embed_ref, b_embed_ref, W1_ref, b1_ref,
                   W2_ref, b2_ref, W_dec_ref, res_ref, out_ref,
                   st_scratch, p1_acc, cs_acc):
    f32 = jnp.float32
    f = W_embed_ref.shape[0]
    D = W_embed_ref.shape[1]
    S = st_scratch.shape[0]
    N = st_scratch.shape[1]
    Wn = N - f
    inv_S = 1.0 / S
    inv_W = 1.0 / Wn
    i = pl.program_id(0)
    nsteps = pl.num_programs(0)
    B = state_ref.shape[0]

    chunk = state_ref[...]

    x_sc = lax.dot_general(chunk[:, :f], W_embed_ref[...],
                           (((1,), (0,)), ((), ())),
                           preferred_element_type=f32) + b_embed_ref[...]
    x_aug = jnp.concatenate([x_sc, jnp.ones((B, 1), f32)], axis=1)
    p1_part = lax.dot_general(chunk, x_aug, (((0,), (0,)), ((), ())),
                              preferred_element_type=f32)
    cs_part = jnp.sum(x_sc, axis=0, keepdims=True)

    @pl.when(i == 0)
    def _init():
        p1_acc[...] = p1_part
        cs_acc[...] = cs_part

    @pl.when(i > 0)
    def _acc():
        p1_acc[...] += p1_part
        cs_acc[...] += cs_part

    st_scratch[pl.ds(i * B, B), :] = chunk

    @pl.when(i == nsteps - 1)
    def _epilogue():
        state = st_scratch[...]
        P1aug = p1_acc[...]
        P1 = P1aug[f:, :D]
        c_col = P1aug[f:, D:]
        colsum_xs = cs_acc[...]

        agg_w1 = (P1 + colsum_xs) * inv_S
        h_w1 = jnp.maximum(
            lax.dot_general(agg_w1, W1_ref[...], (((1,), (0,)), ((), ())),
                            preferred_element_type=f32) + b1_ref[...], 0.0)

        r1 = jnp.maximum(b1_ref[...], 0.0)

        colsum_hw1 = jnp.sum(h_w1, axis=0, keepdims=True)
        hw1_ones = jnp.concatenate([h_w1, jnp.ones((Wn, 1), f32)], axis=1)
        hw1_aug = jnp.concatenate([jnp.zeros((f, D + 1), f32), hw1_ones],
                                  axis=0)
        Qaug = lax.dot_general(state, hw1_aug, (((1,), (0,)), ((), ())),
                               preferred_element_type=f32)
        Q = Qaug[:, :D]
        rowsumA = Qaug[:, D:]
        agg_s2 = (Q + colsum_hw1) * inv_W
        h_s2 = jnp.maximum(
            lax.dot_general(agg_s2, W2_ref[...], (((1,), (0,)), ((), ())),
                            preferred_element_type=f32) + b2_ref[...], 0.0)

        t_row = lax.dot_general(r1, W2_ref[...], (((1,), (0,)), ((), ())),
                                preferred_element_type=f32)
        cscale = 1.0 + c_col * inv_S
        h_w2 = jnp.maximum(cscale * t_row + b2_ref[...], 0.0)

        iota_col = lax.broadcasted_iota(jnp.int32, (S, 1), 0)
        masked = jnp.where(rowsumA == 0.0, iota_col, S)
        si = jnp.min(masked)
        si = jnp.where(si >= S, 0, si)

        onehot = (iota_col == si).astype(f32)
        shift_h = lax.dot_general(onehot, h_s2, (((0,), (0,)), ((), ())),
                                  preferred_element_type=f32)
        v_col = lax.dot_general(W_dec_ref[...], shift_h,
                                (((1,), (1,)), ((), ())),
                                preferred_element_type=f32)
        scores = lax.dot_general(h_w2, v_col, (((1,), (0,)), ((), ())),
                                 preferred_element_type=f32)
        scores = scores + res_ref[0, 0]

        m = jnp.max(scores, axis=0, keepdims=True)
        e = jnp.exp(scores - m)
        out_ref[...] = e / jnp.sum(e, axis=0, keepdims=True)


def kernel(state, W_embed, b_embed, W1, b1, W2, b2, W_dec, count_shifts,
           shift_features):
    f = W_embed.shape[0]
    S = state.shape[0]
    N = state.shape[1]
    Wn = N - f
    D = W_embed.shape[1]
    res = ((jnp.asarray(count_shifts) - S) + (jnp.asarray(shift_features) - f))
    res = res.astype(state.dtype).reshape(1, 1)
    nsteps = S // _CHUNK
    full = lambda i: (0, 0)
    out = pl.pallas_call(
        _policy_kernel,
        grid=(nsteps,),
        in_specs=[
            pl.BlockSpec((_CHUNK, N), lambda i: (i, 0)),
            pl.BlockSpec((f, D), full),
            pl.BlockSpec((1, D), full),
            pl.BlockSpec((D, D), full),
            pl.BlockSpec((1, D), full),
            pl.BlockSpec((D, D), full),
            pl.BlockSpec((1, D), full),
            pl.BlockSpec((D, D), full),
            pl.BlockSpec((1, 1), full),
        ],
        out_specs=pl.BlockSpec((Wn, 1), full),
        scratch_shapes=[
            pltpu.VMEM((S, N), jnp.float32),
            pltpu.VMEM((N, D + 1), jnp.float32),
            pltpu.VMEM((1, D), jnp.float32),
        ],
        out_shape=jax.ShapeDtypeStruct((Wn, 1), state.dtype),
    )(state, W_embed, b_embed.reshape(1, D), W1, b1.reshape(1, D),
      W2, b2.reshape(1, D), W_dec, res)
    return out.reshape(Wn)


# speedup vs baseline: 1.0652x; 1.0652x over previous
import jax
import jax.numpy as jnp
from jax import lax
from jax.experimental import pallas as pl
from jax.experimental.pallas import tpu as pltpu

_NCHUNKS = 5


def _policy_kernel(state_hbm, W_embed_ref, b_embed_ref, W1_ref, b1_ref,
                   W2_ref, b2_ref, W_dec_ref, res_ref, out_ref,
                   st, sems):
    f32 = jnp.float32
    f = W_embed_ref.shape[0]
    D = W_embed_ref.shape[1]
    S = st.shape[0]
    N = st.shape[1]
    Wn = N - f
    inv_S = 1.0 / S
    inv_W = 1.0 / Wn
    B = S // _NCHUNKS

    copies = [
        pltpu.make_async_copy(
            state_hbm.at[pl.ds(k * B, B), :], st.at[pl.ds(k * B, B), :],
            sems.at[k])
        for k in range(_NCHUNKS)
    ]
    for c in copies:
        c.start()

    p1aug = jnp.zeros((N, D + 1), f32)
    colsum_xs = jnp.zeros((1, D), f32)
    for k in range(_NCHUNKS):
        copies[k].wait()
        chunk = st[pl.ds(k * B, B), :]
        x_sc = lax.dot_general(chunk[:, :f], W_embed_ref[...],
                               (((1,), (0,)), ((), ())),
                               preferred_element_type=f32) + b_embed_ref[...]
        x_aug = jnp.concatenate([x_sc, jnp.ones((B, 1), f32)], axis=1)
        p1aug = p1aug + lax.dot_general(chunk, x_aug, (((0,), (0,)), ((), ())),
                                        preferred_element_type=f32)
        colsum_xs = colsum_xs + jnp.sum(x_sc, axis=0, keepdims=True)

    P1 = p1aug[f:, :D]
    c_col = p1aug[f:, D:]

    agg_w1 = (P1 + colsum_xs) * inv_S
    h_w1 = jnp.maximum(
        lax.dot_general(agg_w1, W1_ref[...], (((1,), (0,)), ((), ())),
                        preferred_element_type=f32) + b1_ref[...], 0.0)

    r1 = jnp.maximum(b1_ref[...], 0.0)

    colsum_hw1 = jnp.sum(h_w1, axis=0, keepdims=True)
    hw1_ones = jnp.concatenate([h_w1, jnp.ones((Wn, 1), f32)], axis=1)
    hw1_aug = jnp.concatenate([jnp.zeros((f, D + 1), f32), hw1_ones], axis=0)
    Qaug = lax.dot_general(st[...], hw1_aug, (((1,), (0,)), ((), ())),
                           preferred_element_type=f32)
    Q = Qaug[:, :D]
    rowsumA = Qaug[:, D:]
    agg_s2 = (Q + colsum_hw1) * inv_W
    h_s2 = jnp.maximum(
        lax.dot_general(agg_s2, W2_ref[...], (((1,), (0,)), ((), ())),
                        preferred_element_type=f32) + b2_ref[...], 0.0)

    t_row = lax.dot_general(r1, W2_ref[...], (((1,), (0,)), ((), ())),
                            preferred_element_type=f32)
    cscale = 1.0 + c_col * inv_S
    h_w2 = jnp.maximum(cscale * t_row + b2_ref[...], 0.0)

    iota_col = lax.broadcasted_iota(jnp.int32, (S, 1), 0)
    masked = jnp.where(rowsumA == 0.0, iota_col, S)
    si = jnp.min(masked)
    si = jnp.where(si >= S, 0, si)

    onehot = (iota_col == si).astype(f32)
    shift_h = lax.dot_general(onehot, h_s2, (((0,), (0,)), ((), ())),
                              preferred_element_type=f32)
    v_col = lax.dot_general(W_dec_ref[...], shift_h, (((1,), (1,)), ((), ())),
                            preferred_element_type=f32)
    scores = lax.dot_general(h_w2, v_col, (((1,), (0,)), ((), ())),
                             preferred_element_type=f32)
    scores = scores + res_ref[0, 0]

    m = jnp.max(scores, axis=0, keepdims=True)
    e = jnp.exp(scores - m)
    out_ref[...] = e / jnp.sum(e, axis=0, keepdims=True)


def kernel(state, W_embed, b_embed, W1, b1, W2, b2, W_dec, count_shifts,
           shift_features):
    f = W_embed.shape[0]
    S = state.shape[0]
    N = state.shape[1]
    Wn = N - f
    D = W_embed.shape[1]
    res = ((jnp.asarray(count_shifts) - S) + (jnp.asarray(shift_features) - f))
    res = res.astype(state.dtype).reshape(1, 1)
    out = pl.pallas_call(
        _policy_kernel,
        in_specs=[
            pl.BlockSpec(memory_space=pltpu.MemorySpace.HBM),
            pl.BlockSpec(memory_space=pltpu.MemorySpace.VMEM),
            pl.BlockSpec(memory_space=pltpu.MemorySpace.VMEM),
            pl.BlockSpec(memory_space=pltpu.MemorySpace.VMEM),
            pl.BlockSpec(memory_space=pltpu.MemorySpace.VMEM),
            pl.BlockSpec(memory_space=pltpu.MemorySpace.VMEM),
            pl.BlockSpec(memory_space=pltpu.MemorySpace.VMEM),
            pl.BlockSpec(memory_space=pltpu.MemorySpace.VMEM),
            pl.BlockSpec(memory_space=pltpu.MemorySpace.VMEM),
        ],
        out_specs=pl.BlockSpec(memory_space=pltpu.MemorySpace.VMEM),
        scratch_shapes=[
            pltpu.VMEM((S, N), jnp.float32),
            pltpu.SemaphoreType.DMA((_NCHUNKS,)),
        ],
        out_shape=jax.ShapeDtypeStruct((Wn, 1), state.dtype),
    )(state, W_embed, b_embed.reshape(1, D), W1, b1.reshape(1, D),
      W2, b2.reshape(1, D), W_dec, res)
    return out.reshape(Wn)


# speedup vs baseline: 1.4108x; 1.3244x over previous
import jax
import jax.numpy as jnp
from jax import lax
from jax.experimental import pallas as pl


def _policy_kernel(state_ref, W_embed_ref, b_embed_ref, W1_ref, b1_ref,
                   W2_ref, b2_row_ref, b2_col_ref, W_dec_ref, res_ref, out_ref):
    f32 = jnp.float32
    f = W_embed_ref.shape[0]
    A = state_ref[:, f:]
    S = A.shape[0]
    Wn = A.shape[1]
    inv_S = 1.0 / S
    inv_W = 1.0 / Wn

    x_s = lax.dot_general(state_ref[:, :f], W_embed_ref[...],
                          (((1,), (0,)), ((), ())), preferred_element_type=f32)
    x_s = x_s + b_embed_ref[...]
    colsum_xs = jnp.sum(x_s, axis=0, keepdims=True)

    P1 = lax.dot_general(A, x_s, (((0,), (0,)), ((), ())),
                         preferred_element_type=f32)
    agg_w1 = (P1 + colsum_xs) * inv_S
    h_w1 = jnp.maximum(
        lax.dot_general(agg_w1, W1_ref[...], (((1,), (0,)), ((), ())),
                        preferred_element_type=f32) + b1_ref[...], 0.0)

    r1 = jnp.maximum(b1_ref[...], 0.0)

    c_row = 1.0 + jnp.sum(A, axis=0, keepdims=True) * inv_S
    t_col = lax.dot_general(W2_ref[...], r1, (((0,), (1,)), ((), ())),
                            preferred_element_type=f32)
    h_w2_T = jnp.maximum(t_col * c_row + b2_col_ref[...], 0.0)

    rowsum = jnp.sum(A, axis=1, keepdims=True)
    iota_col = lax.broadcasted_iota(jnp.int32, (S, 1), 0)
    masked = jnp.where(rowsum == 0.0, iota_col, S)
    si = jnp.min(masked)
    si = jnp.where(si >= S, 0, si)

    colsum_hw1 = jnp.sum(h_w1, axis=0, keepdims=True)
    a_row = state_ref[pl.ds(si, 1), :][:, f:]
    u1 = lax.dot_general(a_row, h_w1, (((1,), (0,)), ((), ())),
                         preferred_element_type=f32)
    agg_si = (u1 + colsum_hw1) * inv_W
    shift_h = jnp.maximum(
        lax.dot_general(agg_si, W2_ref[...], (((1,), (0,)), ((), ())),
                        preferred_element_type=f32) + b2_row_ref[...], 0.0)

    v = lax.dot_general(shift_h, W_dec_ref[...], (((1,), (1,)), ((), ())),
                        preferred_element_type=f32)
    scores = lax.dot_general(v, h_w2_T, (((1,), (0,)), ((), ())),
                             preferred_element_type=f32)
    scores = scores + res_ref[0, 0]

    m = jnp.max(scores, axis=1, keepdims=True)
    e = jnp.exp(scores - m)
    out_ref[...] = e / jnp.sum(e, axis=1, keepdims=True)


def kernel(state, W_embed, b_embed, W1, b1, W2, b2, W_dec, count_shifts,
           shift_features):
    f = W_embed.shape[0]
    S = state.shape[0]
    Wn = state.shape[1] - f
    D = W_embed.shape[1]
    res = ((jnp.asarray(count_shifts) - S) + (jnp.asarray(shift_features) - f))
    res = res.astype(state.dtype).reshape(1, 1)
    out = pl.pallas_call(
        _policy_kernel,
        out_shape=jax.ShapeDtypeStruct((1, Wn), state.dtype),
    )(state, W_embed, b_embed.reshape(1, D), W1, b1.reshape(1, D),
      W2, b2.reshape(1, D), b2.reshape(D, 1), W_dec, res)
    return out.reshape(Wn)
